# baseline (device time: 42030 ns/iter reference)
import jax
import jax.numpy as jnp
from jax import lax
from jax.experimental import pallas as pl
from jax.experimental.pallas import tpu as pltpu

TOKENS = 1024
DIM = 1024
VOCAB_PER_X = 8192
BLOCKS = 4
BLK = TOKENS // BLOCKS


def kernel(ids, E):
    my_x = lax.axis_index("x")
    my_y = lax.axis_index("y")
    my_z = lax.axis_index("z")

    blk = my_y * 2 + my_z
    ids_blk = lax.dynamic_slice(ids, (blk * BLK,), (BLK,))
    loc = ids_blk - my_x * VOCAB_PER_X
    mask = (loc >= 0) & (loc < VOCAB_PER_X)
    rows = E[jnp.where(mask, loc, 0)]
    part = jnp.where(mask[:, None], rows, 0.0).astype(jnp.bfloat16)

    def body(part_ref, out_ref, gbuf, xrecv, send_sems, recv_sems):
        x = lax.axis_index("x")
        y = lax.axis_index("y")
        z = lax.axis_index("z")
        xn = (1 - x, y, z)
        yn = (x, 1 - y, z)
        zn = (x, y, 1 - z)

        bar = pltpu.get_barrier_semaphore()
        for nbr in (xn, yn, zn):
            pl.semaphore_signal(
                bar, inc=1, device_id=nbr,
                device_id_type=pl.DeviceIdType.MESH,
            )
        pl.semaphore_wait(bar, 3)

        r1 = pltpu.make_async_remote_copy(
            src_ref=part_ref,
            dst_ref=xrecv,
            send_sem=send_sems.at[0],
            recv_sem=recv_sems.at[0],
            device_id=xn,
            device_id_type=pl.DeviceIdType.MESH,
        )
        r1.start()
        r1.wait()
        gbuf[0, :, :] = part_ref[:, :] + xrecv[:, :]

        r2 = pltpu.make_async_remote_copy(
            src_ref=gbuf.at[0],
            dst_ref=gbuf.at[1],
            send_sem=send_sems.at[1],
            recv_sem=recv_sems.at[1],
            device_id=zn,
            device_id_type=pl.DeviceIdType.MESH,
        )
        r2.start()
        r2.wait()

        r3a = pltpu.make_async_remote_copy(
            src_ref=gbuf.at[0],
            dst_ref=gbuf.at[2],
            send_sem=send_sems.at[2],
            recv_sem=recv_sems.at[2],
            device_id=yn,
            device_id_type=pl.DeviceIdType.MESH,
        )
        r3b = pltpu.make_async_remote_copy(
            src_ref=gbuf.at[1],
            dst_ref=gbuf.at[3],
            send_sem=send_sems.at[3],
            recv_sem=recv_sems.at[3],
            device_id=yn,
            device_id_type=pl.DeviceIdType.MESH,
        )
        r3a.start()
        r3b.start()
        r3a.wait()
        r3b.wait()

        for s, (by, bz) in enumerate(
            [(y, z), (y, 1 - z), (1 - y, z), (1 - y, 1 - z)]
        ):
            row0 = (by * 2 + bz) * BLK
            out_ref[pl.ds(row0, BLK), :] = gbuf[s, :, :].astype(jnp.float32)

    return pl.pallas_call(
        body,
        out_shape=jax.ShapeDtypeStruct((TOKENS, DIM), jnp.float32),
        in_specs=[pl.BlockSpec(memory_space=pltpu.VMEM)],
        out_specs=pl.BlockSpec(memory_space=pltpu.VMEM),
        scratch_shapes=[
            pltpu.VMEM((BLOCKS, BLK, DIM), jnp.bfloat16),
            pltpu.VMEM((BLK, DIM), jnp.bfloat16),
            pltpu.SemaphoreType.DMA((4,)),
            pltpu.SemaphoreType.DMA((4,)),
        ],
        compiler_params=pltpu.CompilerParams(collective_id=0),
    )(part)


# device time: 31719 ns/iter; 1.3251x vs baseline; 1.3251x over previous
import jax
import jax.numpy as jnp
from jax import lax
from jax.experimental import pallas as pl
from jax.experimental.pallas import tpu as pltpu

TOKENS = 1024
DIM = 1024
VOCAB_PER_X = 8192
BLOCKS = 4
BLK = TOKENS // BLOCKS
C = 2
CH = BLK // C


def kernel(ids, E):
    my_x = lax.axis_index("x")
    my_y = lax.axis_index("y")
    my_z = lax.axis_index("z")

    blk = my_y * 2 + my_z
    ids_blk = lax.dynamic_slice(ids, (blk * BLK,), (BLK,))
    loc = ids_blk - my_x * VOCAB_PER_X
    mask = (loc >= 0) & (loc < VOCAB_PER_X)
    rows = E[jnp.where(mask, loc, 0)]
    part = jnp.where(mask[:, None], rows, 0.0).astype(jnp.bfloat16)

    def body(part_ref, out_ref, gbuf, xrecv, send_sems, recv_sems):
        x = lax.axis_index("x")
        y = lax.axis_index("y")
        z = lax.axis_index("z")
        xn = (1 - x, y, z)
        yn = (x, 1 - y, z)
        zn = (x, y, 1 - z)

        def rdma(src, dst, sem, dev):
            return pltpu.make_async_remote_copy(
                src_ref=src, dst_ref=dst,
                send_sem=send_sems.at[sem], recv_sem=recv_sems.at[sem],
                device_id=dev, device_id_type=pl.DeviceIdType.MESH,
            )

        def gslice(s, c):
            return gbuf.at[pl.ds(s * BLK + c * CH, CH), :]

        bar = pltpu.get_barrier_semaphore()
        for nbr in (xn, yn, zn):
            pl.semaphore_signal(
                bar, inc=1, device_id=nbr,
                device_id_type=pl.DeviceIdType.MESH,
            )
        pl.semaphore_wait(bar, 3)

        r1 = [rdma(part_ref.at[pl.ds(c * CH, CH), :],
                   xrecv.at[pl.ds(c * CH, CH), :], c, xn)
              for c in range(C)]
        for r in r1:
            r.start()

        r2 = []
        r3a = []
        for c in range(C):
            r1[c].wait_recv()
            gbuf[pl.ds(c * CH, CH), :] = (
                part_ref[pl.ds(c * CH, CH), :]
                + xrecv[pl.ds(c * CH, CH), :]
            )
            r2.append(rdma(gslice(0, c), gslice(1, c), C + c, zn))
            r3a.append(rdma(gslice(0, c), gslice(2, c), 2 * C + c, yn))
            r2[c].start()
            r3a[c].start()

        out_ref[pl.ds((y * 2 + z) * BLK, BLK), :] = (
            gbuf[pl.ds(0, BLK), :].astype(jnp.float32)
        )

        r3b = []
        for c in range(C):
            r2[c].wait_recv()
            r3b.append(rdma(gslice(1, c), gslice(3, c), 3 * C + c, yn))
            r3b[c].start()

        out_ref[pl.ds((y * 2 + (1 - z)) * BLK, BLK), :] = (
            gbuf[pl.ds(BLK, BLK), :].astype(jnp.float32)
        )

        for c in range(C):
            r3a[c].wait_recv()
        out_ref[pl.ds(((1 - y) * 2 + z) * BLK, BLK), :] = (
            gbuf[pl.ds(2 * BLK, BLK), :].astype(jnp.float32)
        )

        for c in range(C):
            r3b[c].wait_recv()
        out_ref[pl.ds(((1 - y) * 2 + (1 - z)) * BLK, BLK), :] = (
            gbuf[pl.ds(3 * BLK, BLK), :].astype(jnp.float32)
        )

        for r in r1 + r2 + r3a + r3b:
            r.wait_send()

    return pl.pallas_call(
        body,
        out_shape=jax.ShapeDtypeStruct((TOKENS, DIM), jnp.float32),
        in_specs=[pl.BlockSpec(memory_space=pltpu.VMEM)],
        out_specs=pl.BlockSpec(memory_space=pltpu.VMEM),
        scratch_shapes=[
            pltpu.VMEM((BLOCKS * BLK, DIM), jnp.bfloat16),
            pltpu.VMEM((BLK, DIM), jnp.bfloat16),
            pltpu.SemaphoreType.DMA((4 * C,)),
            pltpu.SemaphoreType.DMA((4 * C,)),
        ],
        compiler_params=pltpu.CompilerParams(collective_id=0),
    )(part)


# device time: 29794 ns/iter; 1.4107x vs baseline; 1.0646x over previous
import jax
import jax.numpy as jnp
from jax import lax
from jax.experimental import pallas as pl
from jax.experimental.pallas import tpu as pltpu

TOKENS = 1024
DIM = 1024
VOCAB_PER_X = 8192
BLOCKS = 4
BLK = TOKENS // BLOCKS
C = 4
CH = BLK // C


def kernel(ids, E):
    my_x = lax.axis_index("x")
    my_y = lax.axis_index("y")
    my_z = lax.axis_index("z")

    blk = my_y * 2 + my_z
    ids_blk = lax.dynamic_slice(ids, (blk * BLK,), (BLK,))
    loc = ids_blk - my_x * VOCAB_PER_X
    mask = (loc >= 0) & (loc < VOCAB_PER_X)
    rows = E[jnp.where(mask, loc, 0)]
    part = jnp.where(mask[:, None], rows, 0.0).astype(jnp.bfloat16)

    def body(part_ref, out_ref, xrecv, send_sems, recv_sems):
        x = lax.axis_index("x")
        y = lax.axis_index("y")
        z = lax.axis_index("z")
        xn = (1 - x, y, z)
        yn = (x, 1 - y, z)
        zn = (x, y, 1 - z)

        b_own = (y * 2 + z) * BLK
        b_z = (y * 2 + (1 - z)) * BLK
        b_y = ((1 - y) * 2 + z) * BLK
        b_yz = ((1 - y) * 2 + (1 - z)) * BLK

        def rdma(src, dst, sem, dev):
            return pltpu.make_async_remote_copy(
                src_ref=src, dst_ref=dst,
                send_sem=send_sems.at[sem], recv_sem=recv_sems.at[sem],
                device_id=dev, device_id_type=pl.DeviceIdType.MESH,
            )

        def oslice(base, c):
            return out_ref.at[pl.ds(base + c * CH, CH), :]

        bar = pltpu.get_barrier_semaphore()
        for nbr in (xn, yn, zn):
            pl.semaphore_signal(
                bar, inc=1, device_id=nbr,
                device_id_type=pl.DeviceIdType.MESH,
            )
        pl.semaphore_wait(bar, 3)

        r1 = [rdma(part_ref.at[pl.ds(c * CH, CH), :],
                   xrecv.at[pl.ds(c * CH, CH), :], c, xn)
              for c in range(C)]
        for r in r1:
            r.start()

        r2 = []
        r3a = []
        for c in range(C):
            r1[c].wait_recv()
            out_ref[pl.ds(b_own + c * CH, CH), :] = (
                part_ref[pl.ds(c * CH, CH), :]
                + xrecv[pl.ds(c * CH, CH), :]
            )
            r2.append(rdma(oslice(b_own, c), oslice(b_own, c), C + c, zn))
            r3a.append(rdma(oslice(b_own, c), oslice(b_own, c), 2 * C + c, yn))
            r2[c].start()
            r3a[c].start()

        r3b = []
        for c in range(C):
            r2[c].wait_recv()
            r3b.append(rdma(oslice(b_z, c), oslice(b_z, c), 3 * C + c, yn))
            r3b[c].start()

        for c in range(C):
            r3a[c].wait_recv()
        for c in range(C):
            r3b[c].wait_recv()

        for r in r1 + r2 + r3a + r3b:
            r.wait_send()

        del b_y, b_yz

    return pl.pallas_call(
        body,
        out_shape=jax.ShapeDtypeStruct((TOKENS, DIM), jnp.bfloat16),
        in_specs=[pl.BlockSpec(memory_space=pltpu.VMEM)],
        out_specs=pl.BlockSpec(memory_space=pltpu.VMEM),
        scratch_shapes=[
            pltpu.VMEM((BLK, DIM), jnp.bfloat16),
            pltpu.SemaphoreType.DMA((4 * C,)),
            pltpu.SemaphoreType.DMA((4 * C,)),
        ],
        compiler_params=pltpu.CompilerParams(collective_id=0),
    )(part)


# device time: 26706 ns/iter; 1.5738x vs baseline; 1.1156x over previous
import jax
import jax.numpy as jnp
from jax import lax
from jax.experimental import pallas as pl
from jax.experimental.pallas import tpu as pltpu

TOKENS = 1024
DIM = 1024
VOCAB_PER_X = 8192
BLOCKS = 4
BLK = TOKENS // BLOCKS
C = 4
CH = BLK // C


def kernel(ids, E):
    my_x = lax.axis_index("x")
    my_y = lax.axis_index("y")
    my_z = lax.axis_index("z")

    blk = my_y * 2 + my_z
    ids_blk = lax.dynamic_slice(ids, (blk * BLK,), (BLK,))
    loc = ids_blk - my_x * VOCAB_PER_X
    mask = (loc >= 0) & (loc < VOCAB_PER_X)
    loc_c = jnp.where(mask, loc, 0).astype(jnp.int32)
    maskcol = mask.astype(jnp.bfloat16)[:, None]

    def body(loc_ref, mask_ref, e_ref, out_ref,
             part32, partb, xrecv, gsems, send_sems, recv_sems):
        x = lax.axis_index("x")
        y = lax.axis_index("y")
        z = lax.axis_index("z")
        xn = (1 - x, y, z)
        yn = (x, 1 - y, z)
        zn = (x, y, 1 - z)

        b_own = (y * 2 + z) * BLK
        b_z = (y * 2 + (1 - z)) * BLK

        def rdma(src, dst, sem, dev):
            return pltpu.make_async_remote_copy(
                src_ref=src, dst_ref=dst,
                send_sem=send_sems.at[sem], recv_sem=recv_sems.at[sem],
                device_id=dev, device_id_type=pl.DeviceIdType.MESH,
            )

        def oslice(base, c):
            return out_ref.at[pl.ds(base + c * CH, CH), :]

        gcopies = []
        for c in range(C):
            chunk = []
            for t in range(CH):
                i = c * CH + t
                chunk.append(pltpu.make_async_copy(
                    e_ref.at[loc_ref[i]], part32.at[i], gsems.at[c]))
            gcopies.append(chunk)
        for chunk in gcopies:
            for cp in chunk:
                cp.start()

        bar = pltpu.get_barrier_semaphore()
        for nbr in (xn, yn, zn):
            pl.semaphore_signal(
                bar, inc=1, device_id=nbr,
                device_id_type=pl.DeviceIdType.MESH,
            )
        pl.semaphore_wait(bar, 3)

        r1 = []
        for c in range(C):
            for cp in gcopies[c]:
                cp.wait()
            sl = pl.ds(c * CH, CH)
            partb[sl, :] = (
                part32[sl, :].astype(jnp.bfloat16) * mask_ref[sl, :]
            )
            r1.append(rdma(partb.at[sl, :], xrecv.at[sl, :], c, xn))
            r1[c].start()

        r2 = []
        r3a = []
        for c in range(C):
            r1[c].wait_recv()
            out_ref[pl.ds(b_own + c * CH, CH), :] = (
                partb[pl.ds(c * CH, CH), :]
                + xrecv[pl.ds(c * CH, CH), :]
            )
            r2.append(rdma(oslice(b_own, c), oslice(b_own, c), C + c, zn))
            r3a.append(rdma(oslice(b_own, c), oslice(b_own, c), 2 * C + c, yn))
            r2[c].start()
            r3a[c].start()

        r3b = []
        for c in range(C):
            r2[c].wait_recv()
            r3b.append(rdma(oslice(b_z, c), oslice(b_z, c), 3 * C + c, yn))
            r3b[c].start()

        for c in range(C):
            r3a[c].wait_recv()
        for c in range(C):
            r3b[c].wait_recv()

        for r in r1 + r2 + r3a + r3b:
            r.wait_send()

    return pl.pallas_call(
        body,
        out_shape=jax.ShapeDtypeStruct((TOKENS, DIM), jnp.bfloat16),
        in_specs=[
            pl.BlockSpec(memory_space=pltpu.SMEM),
            pl.BlockSpec(memory_space=pltpu.VMEM),
            pl.BlockSpec(memory_space=pl.ANY),
        ],
        out_specs=pl.BlockSpec(memory_space=pltpu.VMEM),
        scratch_shapes=[
            pltpu.VMEM((BLK, DIM), jnp.float32),
            pltpu.VMEM((BLK, DIM), jnp.bfloat16),
            pltpu.VMEM((BLK, DIM), jnp.bfloat16),
            pltpu.SemaphoreType.DMA((C,)),
            pltpu.SemaphoreType.DMA((4 * C,)),
            pltpu.SemaphoreType.DMA((4 * C,)),
        ],
        compiler_params=pltpu.CompilerParams(collective_id=0),
    )(loc_c, maskcol, E)
